# baseline (speedup 1.0000x reference)
import functools

import jax
import jax.numpy as jnp
from jax import lax
from jax.experimental import pallas as pl
from jax.experimental.pallas import tpu as pltpu
from jax.experimental.pallas import tpu_sc as plsc

N = 10000
E = 320000
D_IN = 128
H = 64
OUT = 32

NW = 32
LN = 128
K0 = 144
K1 = 16
TOTCH = 16 * (K0 + K1)
E_PAD = TOTCH * LN
N_PAD = 10112
RPT = N_PAD // 16

_mesh = plsc.VectorSubcoreMesh(core_axis_name="c", subcore_axis_name="s")


@functools.partial(
    pl.kernel,
    mesh=_mesh,
    compiler_params=pltpu.CompilerParams(use_tc_tiling_on_sc=False),
    out_type=jax.ShapeDtypeStruct((2, N_PAD, H), jnp.float32),
    scratch_types=[
        pltpu.VMEM((K0, LN), jnp.int32),
        pltpu.VMEM((K0, LN), jnp.int32),
        pltpu.VMEM((LN, H), jnp.float32),
        pltpu.VMEM_SHARED((N_PAD, H), jnp.float32),
        pltpu.SemaphoreType.DMA,
    ],
)
def _segsum_sc(rows_hbm, src_hbm, dst_hbm, zeros_hbm, out_hbm,
               src_v, dst_v, buf_v, acc_sh, sem):
    c = lax.axis_index("c")
    s = lax.axis_index("s")

    def body(j, carry):
        pltpu.async_copy(rows_hbm.at[src_v.at[j]], buf_v, sem).wait()
        pltpu.sync_copy(buf_v, acc_sh.at[dst_v.at[j]], add=True)
        return carry

    with jax.named_scope("segsum_stage"):
        pltpu.sync_copy(zeros_hbm.at[pl.ds(s * RPT, RPT)],
                        acc_sh.at[pl.ds(s * RPT, RPT)])
        plsc.subcore_barrier()

    with jax.named_scope("segsum_edges"):
        wid = s * 2 + c
        pltpu.sync_copy(src_hbm.at[wid], src_v)
        pltpu.sync_copy(dst_hbm.at[wid], dst_v)

        @pl.when(c == 0)
        def _():
            lax.fori_loop(0, K0, body, 0)

        @pl.when(c == 1)
        def _():
            lax.fori_loop(0, K1, body, 0)

        plsc.subcore_barrier()

    with jax.named_scope("segsum_drain"):
        pltpu.sync_copy(acc_sh.at[pl.ds(s * RPT, RPT)],
                        out_hbm.at[c, pl.ds(s * RPT, RPT)])


def _mm_body(x_ref, w_ref, o_ref):
    o_ref[...] = jnp.dot(x_ref[...], w_ref[...],
                         preferred_element_type=jnp.float32)


def _mid_body(a_ref, b_ref, w_ref, o_ref):
    m = a_ref[...] + b_ref[...]
    n = jnp.sqrt(jnp.sum(m * m, axis=1, keepdims=True))
    h = jnp.maximum(m / jnp.maximum(n, 1e-12), 0.0)
    o_ref[...] = jnp.dot(h, w_ref[...], preferred_element_type=jnp.float32)


def _head_body(a_ref, b_ref, eps_ref, wm1_ref, bm1_ref, wm2_ref, bm2_ref,
               wl1_ref, bl1_ref, wl2_ref, bl2_ref, o_ref):
    m = a_ref[...] + b_ref[...]
    n = jnp.sqrt(jnp.sum(m * m, axis=1, keepdims=True))
    h = jnp.maximum(m / jnp.maximum(n, 1e-12), 0.0)
    t1 = jnp.tanh(jnp.dot(h, wm1_ref[...], preferred_element_type=jnp.float32)
                  + bm1_ref[...])
    mu = jnp.dot(t1, wm2_ref[...], preferred_element_type=jnp.float32) \
        + bm2_ref[...]
    t2 = jnp.tanh(jnp.dot(h, wl1_ref[...], preferred_element_type=jnp.float32)
                  + bl1_ref[...])
    ls = jnp.dot(t2, wl2_ref[...], preferred_element_type=jnp.float32) \
        + bl2_ref[...]
    ls = jnp.minimum(ls, 10.0)
    o_ref[...] = mu + eps_ref[...] * jnp.exp(ls)


_BM = 1000


def _tc_matmul(x, w):
    return pl.pallas_call(
        _mm_body,
        grid=(N // _BM,),
        in_specs=[
            pl.BlockSpec((_BM, x.shape[1]), lambda i: (i, 0)),
            pl.BlockSpec(w.shape, lambda i: (0, 0)),
        ],
        out_specs=pl.BlockSpec((_BM, w.shape[1]), lambda i: (i, 0)),
        out_shape=jax.ShapeDtypeStruct((N, w.shape[1]), jnp.float32),
    )(x, w)


def _tc_mid(pa, pb, w):
    return pl.pallas_call(
        _mid_body,
        grid=(N // _BM,),
        in_specs=[
            pl.BlockSpec((_BM, H), lambda i: (i, 0)),
            pl.BlockSpec((_BM, H), lambda i: (i, 0)),
            pl.BlockSpec((H, H), lambda i: (0, 0)),
        ],
        out_specs=pl.BlockSpec((_BM, H), lambda i: (i, 0)),
        out_shape=jax.ShapeDtypeStruct((N, H), jnp.float32),
    )(pa, pb, w)


def _tc_head(pa, pb, eps, wm1, bm1, wm2, bm2, wl1, bl1, wl2, bl2):
    full = lambda shape: pl.BlockSpec(shape, lambda i: tuple(0 for _ in shape))
    return pl.pallas_call(
        _head_body,
        grid=(N // _BM,),
        in_specs=[
            pl.BlockSpec((_BM, H), lambda i: (i, 0)),
            pl.BlockSpec((_BM, H), lambda i: (i, 0)),
            pl.BlockSpec((_BM, OUT), lambda i: (i, 0)),
            full(wm1.shape), full((1, OUT // 2)),
            full(wm2.shape), full((1, OUT)),
            full(wl1.shape), full((1, OUT // 2)),
            full(wl2.shape), full((1, OUT)),
        ],
        out_specs=pl.BlockSpec((_BM, OUT), lambda i: (i, 0)),
        out_shape=jax.ShapeDtypeStruct((N, OUT), jnp.float32),
    )(pa, pb, eps, wm1, bm1.reshape(1, -1), wm2, bm2.reshape(1, -1),
      wl1, bl1.reshape(1, -1), wl2, bl2.reshape(1, -1))


def kernel(x, edge_index, eps, W0, W1, Wm1, bm1, Wm2, bm2, Wl1, bl1, Wl2, bl2):
    src = edge_index[0].astype(jnp.int32)
    dst = edge_index[1].astype(jnp.int32)
    pad = E_PAD - E
    flat_src = jnp.concatenate([src, jnp.zeros((pad,), jnp.int32)])
    flat_dst = jnp.concatenate([dst, jnp.full((pad,), N, jnp.int32)])
    flat_src = flat_src.reshape(TOTCH, LN)
    flat_dst = flat_dst.reshape(TOTCH, LN)
    src_p = jnp.zeros((NW, K0, LN), jnp.int32)
    dst_p = jnp.full((NW, K0, LN), N, jnp.int32)
    row = 0
    for t in range(NW):
        cnt = K0 if t % 2 == 0 else K1
        src_p = src_p.at[t, :cnt].set(flat_src[row:row + cnt])
        dst_p = dst_p.at[t, :cnt].set(flat_dst[row:row + cnt])
        row += cnt
    zeros = jnp.zeros((N_PAD, H), jnp.float32)

    s0 = _tc_matmul(x, W0)
    p0 = _segsum_sc(s0, src_p, dst_p, zeros)
    s1 = _tc_mid(p0[0, :N], p0[1, :N], W1)
    p1 = _segsum_sc(s1, src_p, dst_p, zeros)
    return _tc_head(p1[0, :N], p1[1, :N], eps, Wm1, bm1, Wm2, bm2,
                    Wl1, bl1, Wl2, bl2)


# speedup vs baseline: 1.4857x; 1.4857x over previous
import functools

import jax
import jax.numpy as jnp
from jax import lax
from jax.experimental import pallas as pl
from jax.experimental.pallas import tpu as pltpu
from jax.experimental.pallas import tpu_sc as plsc

N = 10000
E = 320000
D_IN = 128
H = 64
OUT = 32

NW = 32
LN = 128
K0 = 79
K1 = 79
TOTCH = 16 * (K0 + K1)
E_PAD = TOTCH * LN
N_PAD = 10112
RPT = N_PAD // 16

_mesh = plsc.VectorSubcoreMesh(core_axis_name="c", subcore_axis_name="s")


@functools.partial(
    pl.kernel,
    mesh=_mesh,
    compiler_params=pltpu.CompilerParams(use_tc_tiling_on_sc=False),
    out_type=jax.ShapeDtypeStruct((2, N_PAD, H), jnp.float32),
    scratch_types=[
        pltpu.VMEM((K0, LN), jnp.int32),
        pltpu.VMEM((K0, LN), jnp.int32),
        pltpu.VMEM((LN, H), jnp.float32),
        pltpu.VMEM_SHARED((N_PAD, H), jnp.float32),
        pltpu.SemaphoreType.DMA,
    ],
)
def _segsum_sc(rows_hbm, src_hbm, dst_hbm, zeros_hbm, out_hbm,
               src_v, dst_v, buf_v, acc_sh, sem):
    c = lax.axis_index("c")
    s = lax.axis_index("s")

    def body(j, carry):
        pltpu.async_copy(rows_hbm.at[src_v.at[j]], buf_v, sem).wait()
        pltpu.sync_copy(buf_v, acc_sh.at[dst_v.at[j]], add=True)
        return carry

    with jax.named_scope("segsum_stage"):
        pltpu.sync_copy(zeros_hbm.at[pl.ds(s * RPT, RPT)],
                        acc_sh.at[pl.ds(s * RPT, RPT)])
        plsc.subcore_barrier()

    with jax.named_scope("segsum_edges"):
        wid = s * 2 + c
        pltpu.sync_copy(src_hbm.at[wid], src_v)
        pltpu.sync_copy(dst_hbm.at[wid], dst_v)

        @pl.when(c == 0)
        def _():
            lax.fori_loop(0, K0, body, 0)

        @pl.when(c == 1)
        def _():
            lax.fori_loop(0, K1, body, 0)

        plsc.subcore_barrier()

    with jax.named_scope("segsum_drain"):
        pltpu.sync_copy(acc_sh.at[pl.ds(s * RPT, RPT)],
                        out_hbm.at[c, pl.ds(s * RPT, RPT)])


def _mm_body(x_ref, w_ref, o_ref):
    o_ref[...] = jnp.dot(x_ref[...], w_ref[...],
                         preferred_element_type=jnp.float32)


def _mid_body(a_ref, b_ref, w_ref, o_ref):
    m = a_ref[...] + b_ref[...]
    n = jnp.sqrt(jnp.sum(m * m, axis=1, keepdims=True))
    h = jnp.maximum(m / jnp.maximum(n, 1e-12), 0.0)
    o_ref[...] = jnp.dot(h, w_ref[...], preferred_element_type=jnp.float32)


def _head_body(a_ref, b_ref, eps_ref, wm1_ref, bm1_ref, wm2_ref, bm2_ref,
               wl1_ref, bl1_ref, wl2_ref, bl2_ref, o_ref):
    m = a_ref[...] + b_ref[...]
    n = jnp.sqrt(jnp.sum(m * m, axis=1, keepdims=True))
    h = jnp.maximum(m / jnp.maximum(n, 1e-12), 0.0)
    t1 = jnp.tanh(jnp.dot(h, wm1_ref[...], preferred_element_type=jnp.float32)
                  + bm1_ref[...])
    mu = jnp.dot(t1, wm2_ref[...], preferred_element_type=jnp.float32) \
        + bm2_ref[...]
    t2 = jnp.tanh(jnp.dot(h, wl1_ref[...], preferred_element_type=jnp.float32)
                  + bl1_ref[...])
    ls = jnp.dot(t2, wl2_ref[...], preferred_element_type=jnp.float32) \
        + bl2_ref[...]
    ls = jnp.minimum(ls, 10.0)
    o_ref[...] = mu + eps_ref[...] * jnp.exp(ls)


_BM = 1000


def _tc_matmul(x, w):
    return pl.pallas_call(
        _mm_body,
        grid=(N // _BM,),
        in_specs=[
            pl.BlockSpec((_BM, x.shape[1]), lambda i: (i, 0)),
            pl.BlockSpec(w.shape, lambda i: (0, 0)),
        ],
        out_specs=pl.BlockSpec((_BM, w.shape[1]), lambda i: (i, 0)),
        out_shape=jax.ShapeDtypeStruct((N, w.shape[1]), jnp.float32),
    )(x, w)


def _tc_mid(pa, pb, w):
    return pl.pallas_call(
        _mid_body,
        grid=(N // _BM,),
        in_specs=[
            pl.BlockSpec((_BM, H), lambda i: (i, 0)),
            pl.BlockSpec((_BM, H), lambda i: (i, 0)),
            pl.BlockSpec((H, H), lambda i: (0, 0)),
        ],
        out_specs=pl.BlockSpec((_BM, H), lambda i: (i, 0)),
        out_shape=jax.ShapeDtypeStruct((N, H), jnp.float32),
    )(pa, pb, w)


def _tc_head(pa, pb, eps, wm1, bm1, wm2, bm2, wl1, bl1, wl2, bl2):
    full = lambda shape: pl.BlockSpec(shape, lambda i: tuple(0 for _ in shape))
    return pl.pallas_call(
        _head_body,
        grid=(N // _BM,),
        in_specs=[
            pl.BlockSpec((_BM, H), lambda i: (i, 0)),
            pl.BlockSpec((_BM, H), lambda i: (i, 0)),
            pl.BlockSpec((_BM, OUT), lambda i: (i, 0)),
            full(wm1.shape), full((1, OUT // 2)),
            full(wm2.shape), full((1, OUT)),
            full(wl1.shape), full((1, OUT // 2)),
            full(wl2.shape), full((1, OUT)),
        ],
        out_specs=pl.BlockSpec((_BM, OUT), lambda i: (i, 0)),
        out_shape=jax.ShapeDtypeStruct((N, OUT), jnp.float32),
    )(pa, pb, eps, wm1, bm1.reshape(1, -1), wm2, bm2.reshape(1, -1),
      wl1, bl1.reshape(1, -1), wl2, bl2.reshape(1, -1))


def kernel(x, edge_index, eps, W0, W1, Wm1, bm1, Wm2, bm2, Wl1, bl1, Wl2, bl2):
    src = edge_index[0].astype(jnp.int32)
    dst = edge_index[1].astype(jnp.int32)
    pad = E_PAD - E
    flat_src = jnp.concatenate([src, jnp.zeros((pad,), jnp.int32)])
    flat_dst = jnp.concatenate([dst, jnp.full((pad,), N, jnp.int32)])
    flat_src = flat_src.reshape(TOTCH, LN)
    flat_dst = flat_dst.reshape(TOTCH, LN)
    src_p = jnp.zeros((NW, K0, LN), jnp.int32)
    dst_p = jnp.full((NW, K0, LN), N, jnp.int32)
    row = 0
    for t in range(NW):
        cnt = K0 if t % 2 == 0 else K1
        src_p = src_p.at[t, :cnt].set(flat_src[row:row + cnt])
        dst_p = dst_p.at[t, :cnt].set(flat_dst[row:row + cnt])
        row += cnt
    zeros = jnp.zeros((N_PAD, H), jnp.float32)

    s0 = _tc_matmul(x, W0)
    p0 = _segsum_sc(s0, src_p, dst_p, zeros)
    s1 = _tc_mid(p0[0, :N], p0[1, :N], W1)
    p1 = _segsum_sc(s1, src_p, dst_p, zeros)
    return _tc_head(p1[0, :N], p1[1, :N], eps, Wm1, bm1, Wm2, bm2,
                    Wl1, bl1, Wl2, bl2)


# speedup vs baseline: 2.2373x; 1.5059x over previous
import functools

import jax
import jax.numpy as jnp
from jax import lax
from jax.experimental import pallas as pl
from jax.experimental.pallas import tpu as pltpu
from jax.experimental.pallas import tpu_sc as plsc

N = 10000
E = 320000
D_IN = 128
H = 64
OUT = 32

NW = 32
LN = 128
CH = 79
TOTCH = NW * CH
E_PAD = TOTCH * LN
N_PAD = 10112
RPT = N_PAD // 16

_mesh = plsc.VectorSubcoreMesh(core_axis_name="c", subcore_axis_name="s")


@functools.partial(
    pl.kernel,
    mesh=_mesh,
    compiler_params=pltpu.CompilerParams(use_tc_tiling_on_sc=False),
    out_type=jax.ShapeDtypeStruct((2, N_PAD, H), jnp.float32),
    scratch_types=[
        pltpu.VMEM((CH, LN), jnp.int32),
        pltpu.VMEM((CH, LN), jnp.int32),
        pltpu.VMEM((LN, H), jnp.float32),
        pltpu.VMEM_SHARED((N_PAD, H), jnp.float32),
        pltpu.VMEM_SHARED((N_PAD, H), jnp.float32),
        pltpu.SemaphoreType.DMA,
    ],
)
def _segsum_sc(rows_hbm, src_hbm, dst_hbm, zeros_hbm, out_hbm,
               src_v, dst_v, buf_v, acc_sh, tbl_sh, sem):
    c = lax.axis_index("c")
    s = lax.axis_index("s")

    def body(j, carry):
        pltpu.async_copy(tbl_sh.at[src_v.at[j]], buf_v, sem).wait()
        pltpu.sync_copy(buf_v, acc_sh.at[dst_v.at[j]], add=True)
        return carry

    with jax.named_scope("segsum_stage"):
        pltpu.sync_copy(rows_hbm.at[pl.ds(s * RPT, RPT)],
                        tbl_sh.at[pl.ds(s * RPT, RPT)])
        pltpu.sync_copy(zeros_hbm.at[pl.ds(s * RPT, RPT)],
                        acc_sh.at[pl.ds(s * RPT, RPT)])
        wid = s * 2 + c
        pltpu.sync_copy(src_hbm.at[wid], src_v)
        pltpu.sync_copy(dst_hbm.at[wid], dst_v)
        plsc.subcore_barrier()

    with jax.named_scope("segsum_edges"):
        lax.fori_loop(0, CH, body, 0)
        plsc.subcore_barrier()

    with jax.named_scope("segsum_drain"):
        pltpu.sync_copy(acc_sh.at[pl.ds(s * RPT, RPT)],
                        out_hbm.at[c, pl.ds(s * RPT, RPT)])


def _mm_body(x_ref, w_ref, o_ref):
    o_ref[...] = jnp.dot(x_ref[...], w_ref[...],
                         preferred_element_type=jnp.float32)


def _mid_body(a_ref, b_ref, w_ref, o_ref):
    m = a_ref[...] + b_ref[...]
    n = jnp.sqrt(jnp.sum(m * m, axis=1, keepdims=True))
    h = jnp.maximum(m / jnp.maximum(n, 1e-12), 0.0)
    o_ref[...] = jnp.dot(h, w_ref[...], preferred_element_type=jnp.float32)


def _head_body(a_ref, b_ref, eps_ref, wm1_ref, bm1_ref, wm2_ref, bm2_ref,
               wl1_ref, bl1_ref, wl2_ref, bl2_ref, o_ref):
    m = a_ref[...] + b_ref[...]
    n = jnp.sqrt(jnp.sum(m * m, axis=1, keepdims=True))
    h = jnp.maximum(m / jnp.maximum(n, 1e-12), 0.0)
    t1 = jnp.tanh(jnp.dot(h, wm1_ref[...], preferred_element_type=jnp.float32)
                  + bm1_ref[...])
    mu = jnp.dot(t1, wm2_ref[...], preferred_element_type=jnp.float32) \
        + bm2_ref[...]
    t2 = jnp.tanh(jnp.dot(h, wl1_ref[...], preferred_element_type=jnp.float32)
                  + bl1_ref[...])
    ls = jnp.dot(t2, wl2_ref[...], preferred_element_type=jnp.float32) \
        + bl2_ref[...]
    ls = jnp.minimum(ls, 10.0)
    o_ref[...] = mu + eps_ref[...] * jnp.exp(ls)


_BM = 1000
_BMP = 632


def _tc_matmul(x, w):
    return pl.pallas_call(
        _mm_body,
        grid=(N_PAD // _BMP,),
        in_specs=[
            pl.BlockSpec((_BMP, x.shape[1]), lambda i: (i, 0)),
            pl.BlockSpec(w.shape, lambda i: (0, 0)),
        ],
        out_specs=pl.BlockSpec((_BMP, w.shape[1]), lambda i: (i, 0)),
        out_shape=jax.ShapeDtypeStruct((N_PAD, w.shape[1]), jnp.float32),
    )(x, w)


def _tc_mid(pa, pb, w):
    return pl.pallas_call(
        _mid_body,
        grid=(N_PAD // _BMP,),
        in_specs=[
            pl.BlockSpec((_BMP, H), lambda i: (i, 0)),
            pl.BlockSpec((_BMP, H), lambda i: (i, 0)),
            pl.BlockSpec((H, H), lambda i: (0, 0)),
        ],
        out_specs=pl.BlockSpec((_BMP, H), lambda i: (i, 0)),
        out_shape=jax.ShapeDtypeStruct((N_PAD, H), jnp.float32),
    )(pa, pb, w)


def _tc_head(pa, pb, eps, wm1, bm1, wm2, bm2, wl1, bl1, wl2, bl2):
    full = lambda shape: pl.BlockSpec(shape, lambda i: tuple(0 for _ in shape))
    return pl.pallas_call(
        _head_body,
        grid=(N // _BM,),
        in_specs=[
            pl.BlockSpec((_BM, H), lambda i: (i, 0)),
            pl.BlockSpec((_BM, H), lambda i: (i, 0)),
            pl.BlockSpec((_BM, OUT), lambda i: (i, 0)),
            full(wm1.shape), full((1, OUT // 2)),
            full(wm2.shape), full((1, OUT)),
            full(wl1.shape), full((1, OUT // 2)),
            full(wl2.shape), full((1, OUT)),
        ],
        out_specs=pl.BlockSpec((_BM, OUT), lambda i: (i, 0)),
        out_shape=jax.ShapeDtypeStruct((N, OUT), jnp.float32),
    )(pa, pb, eps, wm1, bm1.reshape(1, -1), wm2, bm2.reshape(1, -1),
      wl1, bl1.reshape(1, -1), wl2, bl2.reshape(1, -1))


def kernel(x, edge_index, eps, W0, W1, Wm1, bm1, Wm2, bm2, Wl1, bl1, Wl2, bl2):
    src = edge_index[0].astype(jnp.int32)
    dst = edge_index[1].astype(jnp.int32)
    pad = E_PAD - E
    src_p = jnp.concatenate([src, jnp.zeros((pad,), jnp.int32)])
    dst_p = jnp.concatenate([dst, jnp.full((pad,), N, jnp.int32)])
    src_p = src_p.reshape(NW, CH, LN)
    dst_p = dst_p.reshape(NW, CH, LN)
    zeros = jnp.zeros((N_PAD, H), jnp.float32)

    s0 = _tc_matmul(x, W0)
    p0 = _segsum_sc(s0, src_p, dst_p, zeros)
    s1 = _tc_mid(p0[0], p0[1], W1)
    p1 = _segsum_sc(s1, src_p, dst_p, zeros)
    return _tc_head(p1[0], p1[1], eps, Wm1, bm1, Wm2, bm2,
                    Wl1, bl1, Wl2, bl2)


# speedup vs baseline: 2.3351x; 1.0437x over previous
import functools

import jax
import jax.numpy as jnp
from jax import lax
from jax.experimental import pallas as pl
from jax.experimental.pallas import tpu as pltpu
from jax.experimental.pallas import tpu_sc as plsc

N = 10000
E = 320000
D_IN = 128
H = 64
OUT = 32

NW = 32
LN = 128
GS = 3
NG = 27
CH = NG * GS
TOTCH = NW * CH
E_PAD = TOTCH * LN
N_PAD = 10112
RPT = N_PAD // 16

_mesh = plsc.VectorSubcoreMesh(core_axis_name="c", subcore_axis_name="s")


@functools.partial(
    pl.kernel,
    mesh=_mesh,
    compiler_params=pltpu.CompilerParams(use_tc_tiling_on_sc=False),
    out_type=jax.ShapeDtypeStruct((2, N_PAD, H), jnp.float32),
    scratch_types=[
        pltpu.VMEM((CH, LN), jnp.int32),
        pltpu.VMEM((CH, LN), jnp.int32),
        pltpu.VMEM((GS, LN, H), jnp.float32),
        pltpu.VMEM_SHARED((N_PAD, H), jnp.float32),
        pltpu.VMEM_SHARED((N_PAD, H), jnp.float32),
        pltpu.SemaphoreType.DMA((GS,)),
        pltpu.SemaphoreType.DMA,
    ],
)
def _segsum_sc(rows_hbm, src_hbm, dst_hbm, zeros_hbm, out_hbm,
               src_v, dst_v, buf_v, acc_sh, tbl_sh, gsem, ssem):
    c = lax.axis_index("c")
    s = lax.axis_index("s")

    def body(g, carry):
        base = g * GS
        gathers = [
            pltpu.async_copy(tbl_sh.at[src_v.at[base + b]], buf_v.at[b],
                             gsem.at[b])
            for b in range(GS)
        ]
        scatters = []
        for b in range(GS):
            gathers[b].wait()
            scatters.append(
                pltpu.async_copy(buf_v.at[b], acc_sh.at[dst_v.at[base + b]],
                                 ssem, add=True))
        for sc in scatters:
            sc.wait()
        return carry

    with jax.named_scope("segsum_stage"):
        pltpu.sync_copy(rows_hbm.at[pl.ds(s * RPT, RPT)],
                        tbl_sh.at[pl.ds(s * RPT, RPT)])
        pltpu.sync_copy(zeros_hbm.at[pl.ds(s * RPT, RPT)],
                        acc_sh.at[pl.ds(s * RPT, RPT)])
        wid = s * 2 + c
        pltpu.sync_copy(src_hbm.at[wid], src_v)
        pltpu.sync_copy(dst_hbm.at[wid], dst_v)
        plsc.subcore_barrier()

    with jax.named_scope("segsum_edges"):
        lax.fori_loop(0, NG, body, 0)
        plsc.subcore_barrier()

    with jax.named_scope("segsum_drain"):
        pltpu.sync_copy(acc_sh.at[pl.ds(s * RPT, RPT)],
                        out_hbm.at[c, pl.ds(s * RPT, RPT)])


def _mm_body(x_ref, w_ref, o_ref):
    o_ref[...] = jnp.dot(x_ref[...], w_ref[...],
                         preferred_element_type=jnp.float32)


def _mid_body(a_ref, b_ref, w_ref, o_ref):
    m = a_ref[...] + b_ref[...]
    n = jnp.sqrt(jnp.sum(m * m, axis=1, keepdims=True))
    h = jnp.maximum(m / jnp.maximum(n, 1e-12), 0.0)
    o_ref[...] = jnp.dot(h, w_ref[...], preferred_element_type=jnp.float32)


def _head_body(a_ref, b_ref, eps_ref, wm1_ref, bm1_ref, wm2_ref, bm2_ref,
               wl1_ref, bl1_ref, wl2_ref, bl2_ref, o_ref):
    m = a_ref[...] + b_ref[...]
    n = jnp.sqrt(jnp.sum(m * m, axis=1, keepdims=True))
    h = jnp.maximum(m / jnp.maximum(n, 1e-12), 0.0)
    t1 = jnp.tanh(jnp.dot(h, wm1_ref[...], preferred_element_type=jnp.float32)
                  + bm1_ref[...])
    mu = jnp.dot(t1, wm2_ref[...], preferred_element_type=jnp.float32) \
        + bm2_ref[...]
    t2 = jnp.tanh(jnp.dot(h, wl1_ref[...], preferred_element_type=jnp.float32)
                  + bl1_ref[...])
    ls = jnp.dot(t2, wl2_ref[...], preferred_element_type=jnp.float32) \
        + bl2_ref[...]
    ls = jnp.minimum(ls, 10.0)
    o_ref[...] = mu + eps_ref[...] * jnp.exp(ls)


_BM = 1000
_BMP = 632


def _tc_matmul(x, w):
    return pl.pallas_call(
        _mm_body,
        grid=(N_PAD // _BMP,),
        in_specs=[
            pl.BlockSpec((_BMP, x.shape[1]), lambda i: (i, 0)),
            pl.BlockSpec(w.shape, lambda i: (0, 0)),
        ],
        out_specs=pl.BlockSpec((_BMP, w.shape[1]), lambda i: (i, 0)),
        out_shape=jax.ShapeDtypeStruct((N_PAD, w.shape[1]), jnp.float32),
    )(x, w)


def _tc_mid(pa, pb, w):
    return pl.pallas_call(
        _mid_body,
        grid=(N_PAD // _BMP,),
        in_specs=[
            pl.BlockSpec((_BMP, H), lambda i: (i, 0)),
            pl.BlockSpec((_BMP, H), lambda i: (i, 0)),
            pl.BlockSpec((H, H), lambda i: (0, 0)),
        ],
        out_specs=pl.BlockSpec((_BMP, H), lambda i: (i, 0)),
        out_shape=jax.ShapeDtypeStruct((N_PAD, H), jnp.float32),
    )(pa, pb, w)


def _tc_head(pa, pb, eps, wm1, bm1, wm2, bm2, wl1, bl1, wl2, bl2):
    full = lambda shape: pl.BlockSpec(shape, lambda i: tuple(0 for _ in shape))
    return pl.pallas_call(
        _head_body,
        grid=(N // _BM,),
        in_specs=[
            pl.BlockSpec((_BM, H), lambda i: (i, 0)),
            pl.BlockSpec((_BM, H), lambda i: (i, 0)),
            pl.BlockSpec((_BM, OUT), lambda i: (i, 0)),
            full(wm1.shape), full((1, OUT // 2)),
            full(wm2.shape), full((1, OUT)),
            full(wl1.shape), full((1, OUT // 2)),
            full(wl2.shape), full((1, OUT)),
        ],
        out_specs=pl.BlockSpec((_BM, OUT), lambda i: (i, 0)),
        out_shape=jax.ShapeDtypeStruct((N, OUT), jnp.float32),
    )(pa, pb, eps, wm1, bm1.reshape(1, -1), wm2, bm2.reshape(1, -1),
      wl1, bl1.reshape(1, -1), wl2, bl2.reshape(1, -1))


def kernel(x, edge_index, eps, W0, W1, Wm1, bm1, Wm2, bm2, Wl1, bl1, Wl2, bl2):
    src = edge_index[0].astype(jnp.int32)
    dst = edge_index[1].astype(jnp.int32)
    pad = E_PAD - E
    src_p = jnp.concatenate([src, jnp.zeros((pad,), jnp.int32)])
    dst_p = jnp.concatenate([dst, jnp.full((pad,), N, jnp.int32)])
    src_p = src_p.reshape(NW, CH, LN)
    dst_p = dst_p.reshape(NW, CH, LN)
    zeros = jnp.zeros((N_PAD, H), jnp.float32)

    s0 = _tc_matmul(x, W0)
    p0 = _segsum_sc(s0, src_p, dst_p, zeros)
    s1 = _tc_mid(p0[0], p0[1], W1)
    p1 = _segsum_sc(s1, src_p, dst_p, zeros)
    return _tc_head(p1[0], p1[1], eps, Wm1, bm1, Wm2, bm2,
                    Wl1, bl1, Wl2, bl2)


# speedup vs baseline: 2.4098x; 1.0320x over previous
import functools

import jax
import jax.numpy as jnp
from jax import lax
from jax.experimental import pallas as pl
from jax.experimental.pallas import tpu as pltpu
from jax.experimental.pallas import tpu_sc as plsc

N = 10000
E = 320000
D_IN = 128
H = 64
OUT = 32

NW = 32
LN = 128
GS = 3
NG0 = 31
NG1 = 23
K0 = NG0 * GS
K1 = NG1 * GS
TOTCH = 16 * (K0 + K1)
E_PAD = TOTCH * LN
N_PAD = 10112
RPT = N_PAD // 16

_mesh = plsc.VectorSubcoreMesh(core_axis_name="c", subcore_axis_name="s")


@functools.partial(
    pl.kernel,
    mesh=_mesh,
    compiler_params=pltpu.CompilerParams(use_tc_tiling_on_sc=False),
    out_type=jax.ShapeDtypeStruct((2, N_PAD, H), jnp.float32),
    scratch_types=[
        pltpu.VMEM((K0, LN), jnp.int32),
        pltpu.VMEM((K0, LN), jnp.int32),
        pltpu.VMEM((GS, LN, H), jnp.float32),
        pltpu.VMEM_SHARED((N_PAD, H), jnp.float32),
        pltpu.VMEM_SHARED((N_PAD, H), jnp.float32),
        pltpu.SemaphoreType.DMA((GS,)),
        pltpu.SemaphoreType.DMA,
    ],
)
def _segsum_sc(rows_hbm, src_hbm, dst_hbm, zeros_hbm, out_hbm,
               src_v, dst_v, buf_v, acc_sh, tbl_sh, gsem, ssem):
    c = lax.axis_index("c")
    s = lax.axis_index("s")

    def body(g, carry):
        base = g * GS
        gathers = [
            pltpu.async_copy(tbl_sh.at[src_v.at[base + b]], buf_v.at[b],
                             gsem.at[b])
            for b in range(GS)
        ]
        scatters = []
        for b in range(GS):
            gathers[b].wait()
            scatters.append(
                pltpu.async_copy(buf_v.at[b], acc_sh.at[dst_v.at[base + b]],
                                 ssem, add=True))
        for sc in scatters:
            sc.wait()
        return carry

    with jax.named_scope("segsum_stage"):
        wid = s * 2 + c
        stages = [
            pltpu.async_copy(rows_hbm.at[pl.ds(s * RPT, RPT)],
                             tbl_sh.at[pl.ds(s * RPT, RPT)], ssem),
            pltpu.async_copy(zeros_hbm.at[pl.ds(s * RPT, RPT)],
                             acc_sh.at[pl.ds(s * RPT, RPT)], ssem),
            pltpu.async_copy(src_hbm.at[wid], src_v, ssem),
            pltpu.async_copy(dst_hbm.at[wid], dst_v, ssem),
        ]
        for st in stages:
            st.wait()
        plsc.subcore_barrier()

    with jax.named_scope("segsum_edges"):
        @pl.when(c == 0)
        def _():
            lax.fori_loop(0, NG0, body, 0)

        @pl.when(c == 1)
        def _():
            lax.fori_loop(0, NG1, body, 0)

        plsc.subcore_barrier()

    with jax.named_scope("segsum_drain"):
        pltpu.sync_copy(acc_sh.at[pl.ds(s * RPT, RPT)],
                        out_hbm.at[c, pl.ds(s * RPT, RPT)])


def _mm_body(x_ref, w_ref, o_ref):
    o_ref[...] = jnp.dot(x_ref[...], w_ref[...],
                         preferred_element_type=jnp.float32)


def _mid_body(a_ref, b_ref, w_ref, o_ref):
    m = a_ref[...] + b_ref[...]
    n = jnp.sqrt(jnp.sum(m * m, axis=1, keepdims=True))
    h = jnp.maximum(m / jnp.maximum(n, 1e-12), 0.0)
    o_ref[...] = jnp.dot(h, w_ref[...], preferred_element_type=jnp.float32)


def _head_body(a_ref, b_ref, eps_ref, wm1_ref, bm1_ref, wm2_ref, bm2_ref,
               wl1_ref, bl1_ref, wl2_ref, bl2_ref, o_ref):
    m = a_ref[...] + b_ref[...]
    n = jnp.sqrt(jnp.sum(m * m, axis=1, keepdims=True))
    h = jnp.maximum(m / jnp.maximum(n, 1e-12), 0.0)
    t1 = jnp.tanh(jnp.dot(h, wm1_ref[...], preferred_element_type=jnp.float32)
                  + bm1_ref[...])
    mu = jnp.dot(t1, wm2_ref[...], preferred_element_type=jnp.float32) \
        + bm2_ref[...]
    t2 = jnp.tanh(jnp.dot(h, wl1_ref[...], preferred_element_type=jnp.float32)
                  + bl1_ref[...])
    ls = jnp.dot(t2, wl2_ref[...], preferred_element_type=jnp.float32) \
        + bl2_ref[...]
    ls = jnp.minimum(ls, 10.0)
    o_ref[...] = mu + eps_ref[...] * jnp.exp(ls)


_BM = 1000
_BMP = 632


def _tc_matmul(x, w):
    return pl.pallas_call(
        _mm_body,
        grid=(N_PAD // _BMP,),
        in_specs=[
            pl.BlockSpec((_BMP, x.shape[1]), lambda i: (i, 0)),
            pl.BlockSpec(w.shape, lambda i: (0, 0)),
        ],
        out_specs=pl.BlockSpec((_BMP, w.shape[1]), lambda i: (i, 0)),
        out_shape=jax.ShapeDtypeStruct((N_PAD, w.shape[1]), jnp.float32),
    )(x, w)


def _tc_mid(pa, pb, w):
    return pl.pallas_call(
        _mid_body,
        grid=(N_PAD // _BMP,),
        in_specs=[
            pl.BlockSpec((_BMP, H), lambda i: (i, 0)),
            pl.BlockSpec((_BMP, H), lambda i: (i, 0)),
            pl.BlockSpec((H, H), lambda i: (0, 0)),
        ],
        out_specs=pl.BlockSpec((_BMP, H), lambda i: (i, 0)),
        out_shape=jax.ShapeDtypeStruct((N_PAD, H), jnp.float32),
    )(pa, pb, w)


def _tc_head(pa, pb, eps, wm1, bm1, wm2, bm2, wl1, bl1, wl2, bl2):
    full = lambda shape: pl.BlockSpec(shape, lambda i: tuple(0 for _ in shape))
    return pl.pallas_call(
        _head_body,
        grid=(N // _BM,),
        in_specs=[
            pl.BlockSpec((_BM, H), lambda i: (i, 0)),
            pl.BlockSpec((_BM, H), lambda i: (i, 0)),
            pl.BlockSpec((_BM, OUT), lambda i: (i, 0)),
            full(wm1.shape), full((1, OUT // 2)),
            full(wm2.shape), full((1, OUT)),
            full(wl1.shape), full((1, OUT // 2)),
            full(wl2.shape), full((1, OUT)),
        ],
        out_specs=pl.BlockSpec((_BM, OUT), lambda i: (i, 0)),
        out_shape=jax.ShapeDtypeStruct((N, OUT), jnp.float32),
    )(pa, pb, eps, wm1, bm1.reshape(1, -1), wm2, bm2.reshape(1, -1),
      wl1, bl1.reshape(1, -1), wl2, bl2.reshape(1, -1))


def kernel(x, edge_index, eps, W0, W1, Wm1, bm1, Wm2, bm2, Wl1, bl1, Wl2, bl2):
    src = edge_index[0].astype(jnp.int32)
    dst = edge_index[1].astype(jnp.int32)
    pad = E_PAD - E
    flat_src = jnp.concatenate([src, jnp.zeros((pad,), jnp.int32)])
    flat_dst = jnp.concatenate([dst, jnp.full((pad,), N, jnp.int32)])
    flat_src = flat_src.reshape(TOTCH, LN)
    flat_dst = flat_dst.reshape(TOTCH, LN)
    src_p = jnp.zeros((NW, K0, LN), jnp.int32)
    dst_p = jnp.full((NW, K0, LN), N, jnp.int32)
    row = 0
    for t in range(NW):
        cnt = K0 if t % 2 == 0 else K1
        src_p = src_p.at[t, :cnt].set(flat_src[row:row + cnt])
        dst_p = dst_p.at[t, :cnt].set(flat_dst[row:row + cnt])
        row += cnt
    zeros = jnp.zeros((N_PAD, H), jnp.float32)

    s0 = _tc_matmul(x, W0)
    p0 = _segsum_sc(s0, src_p, dst_p, zeros)
    s1 = _tc_mid(p0[0], p0[1], W1)
    p1 = _segsum_sc(s1, src_p, dst_p, zeros)
    return _tc_head(p1[0], p1[1], eps, Wm1, bm1, Wm2, bm2,
                    Wl1, bl1, Wl2, bl2)


# speedup vs baseline: 2.5248x; 1.0477x over previous
import functools

import jax
import jax.numpy as jnp
from jax import lax
from jax.experimental import pallas as pl
from jax.experimental.pallas import tpu as pltpu
from jax.experimental.pallas import tpu_sc as plsc

N = 10000
E = 320000
D_IN = 128
H = 64
OUT = 32

NW = 32
LN = 128
GS = 3
NG0 = 31
NG1 = 23
K0 = NG0 * GS
K1 = NG1 * GS
TOTCH = 16 * (K0 + K1)
E_PAD = TOTCH * LN
N_PAD = 10112
RPT = N_PAD // 16

_mesh = plsc.VectorSubcoreMesh(core_axis_name="c", subcore_axis_name="s")


@functools.partial(
    pl.kernel,
    mesh=_mesh,
    compiler_params=pltpu.CompilerParams(use_tc_tiling_on_sc=False),
    out_type=jax.ShapeDtypeStruct((2, N_PAD, H), jnp.float32),
    scratch_types=[
        pltpu.VMEM((K0, LN), jnp.int32),
        pltpu.VMEM((K0, LN), jnp.int32),
        pltpu.VMEM((GS, LN, H), jnp.float32),
        pltpu.VMEM_SHARED((N_PAD, H), jnp.float32),
        pltpu.VMEM_SHARED((N_PAD, H), jnp.float32),
        pltpu.SemaphoreType.DMA((GS,)),
        pltpu.SemaphoreType.DMA,
    ],
)
def _segsum_sc(rows_hbm, src_hbm, dst_hbm, zeros_hbm, out_hbm,
               src_v, dst_v, buf_v, acc_sh, tbl_sh, gsem, ssem):
    c = lax.axis_index("c")
    s = lax.axis_index("s")

    def body(g, carry):
        base = g * GS
        gathers = [
            pltpu.async_copy(tbl_sh.at[src_v.at[base + b]], buf_v.at[b],
                             gsem.at[b])
            for b in range(GS)
        ]
        scatters = []
        for b in range(GS):
            gathers[b].wait()
            scatters.append(
                pltpu.async_copy(buf_v.at[b], acc_sh.at[dst_v.at[base + b]],
                                 ssem, add=True))
        for sc in scatters:
            sc.wait()
        return carry

    with jax.named_scope("segsum_stage"):
        wid = s * 2 + c
        stages = [
            pltpu.async_copy(rows_hbm.at[pl.ds(s * RPT, RPT)],
                             tbl_sh.at[pl.ds(s * RPT, RPT)], ssem),
            pltpu.async_copy(zeros_hbm.at[pl.ds(s * RPT, RPT)],
                             acc_sh.at[pl.ds(s * RPT, RPT)], ssem),
            pltpu.async_copy(src_hbm.at[wid], src_v, ssem),
            pltpu.async_copy(dst_hbm.at[wid], dst_v, ssem),
        ]
        for st in stages:
            st.wait()
        plsc.subcore_barrier()

    with jax.named_scope("segsum_edges"):
        @pl.when(c == 0)
        def _():
            lax.fori_loop(0, NG0, body, 0)

        @pl.when(c == 1)
        def _():
            lax.fori_loop(0, NG1, body, 0)

        plsc.subcore_barrier()

    with jax.named_scope("segsum_drain"):
        pltpu.sync_copy(acc_sh.at[pl.ds(s * RPT, RPT)],
                        out_hbm.at[c, pl.ds(s * RPT, RPT)])


def _mm_body(x_ref, w_ref, o_ref):
    o_ref[...] = jnp.dot(x_ref[...], w_ref[...],
                         preferred_element_type=jnp.float32)


def _mid_body(p_ref, w_ref, o_ref):
    m = p_ref[0] + p_ref[1]
    n = jnp.sqrt(jnp.sum(m * m, axis=1, keepdims=True))
    h = jnp.maximum(m / jnp.maximum(n, 1e-12), 0.0)
    o_ref[...] = jnp.dot(h, w_ref[...], preferred_element_type=jnp.float32)


def _head_body(p_ref, eps_ref, wm1_ref, bm1_ref, wm2_ref, bm2_ref,
               wl1_ref, bl1_ref, wl2_ref, bl2_ref, o_ref):
    m = p_ref[0] + p_ref[1]
    n = jnp.sqrt(jnp.sum(m * m, axis=1, keepdims=True))
    h = jnp.maximum(m / jnp.maximum(n, 1e-12), 0.0)
    t1 = jnp.tanh(jnp.dot(h, wm1_ref[...], preferred_element_type=jnp.float32)
                  + bm1_ref[...])
    mu = jnp.dot(t1, wm2_ref[...], preferred_element_type=jnp.float32) \
        + bm2_ref[...]
    t2 = jnp.tanh(jnp.dot(h, wl1_ref[...], preferred_element_type=jnp.float32)
                  + bl1_ref[...])
    ls = jnp.dot(t2, wl2_ref[...], preferred_element_type=jnp.float32) \
        + bl2_ref[...]
    ls = jnp.minimum(ls, 10.0)
    o_ref[...] = mu + eps_ref[...] * jnp.exp(ls)


_BM = 1000
_BMP = 632


def _tc_matmul(x, w):
    return pl.pallas_call(
        _mm_body,
        grid=(N_PAD // _BMP,),
        in_specs=[
            pl.BlockSpec((_BMP, x.shape[1]), lambda i: (i, 0)),
            pl.BlockSpec(w.shape, lambda i: (0, 0)),
        ],
        out_specs=pl.BlockSpec((_BMP, w.shape[1]), lambda i: (i, 0)),
        out_shape=jax.ShapeDtypeStruct((N_PAD, w.shape[1]), jnp.float32),
    )(x, w)


def _tc_mid(p, w):
    return pl.pallas_call(
        _mid_body,
        grid=(N_PAD // _BMP,),
        in_specs=[
            pl.BlockSpec((2, _BMP, H), lambda i: (0, i, 0)),
            pl.BlockSpec((H, H), lambda i: (0, 0)),
        ],
        out_specs=pl.BlockSpec((_BMP, H), lambda i: (i, 0)),
        out_shape=jax.ShapeDtypeStruct((N_PAD, H), jnp.float32),
    )(p, w)


def _tc_head(p, eps, wm1, bm1, wm2, bm2, wl1, bl1, wl2, bl2):
    full = lambda shape: pl.BlockSpec(shape, lambda i: tuple(0 for _ in shape))
    return pl.pallas_call(
        _head_body,
        grid=(N // _BM,),
        in_specs=[
            pl.BlockSpec((2, _BM, H), lambda i: (0, i, 0)),
            pl.BlockSpec((_BM, OUT), lambda i: (i, 0)),
            full(wm1.shape), full((1, OUT // 2)),
            full(wm2.shape), full((1, OUT)),
            full(wl1.shape), full((1, OUT // 2)),
            full(wl2.shape), full((1, OUT)),
        ],
        out_specs=pl.BlockSpec((_BM, OUT), lambda i: (i, 0)),
        out_shape=jax.ShapeDtypeStruct((N, OUT), jnp.float32),
    )(p, eps, wm1, bm1.reshape(1, -1), wm2, bm2.reshape(1, -1),
      wl1, bl1.reshape(1, -1), wl2, bl2.reshape(1, -1))


def kernel(x, edge_index, eps, W0, W1, Wm1, bm1, Wm2, bm2, Wl1, bl1, Wl2, bl2):
    src = edge_index[0].astype(jnp.int32)
    dst = edge_index[1].astype(jnp.int32)
    pad = E_PAD - E
    flat_src = jnp.concatenate([src, jnp.zeros((pad,), jnp.int32)])
    flat_dst = jnp.concatenate([dst, jnp.full((pad,), N, jnp.int32)])
    flat_src = flat_src.reshape(TOTCH, LN)
    flat_dst = flat_dst.reshape(TOTCH, LN)
    src_p = jnp.zeros((NW, K0, LN), jnp.int32)
    dst_p = jnp.full((NW, K0, LN), N, jnp.int32)
    row = 0
    for t in range(NW):
        cnt = K0 if t % 2 == 0 else K1
        src_p = src_p.at[t, :cnt].set(flat_src[row:row + cnt])
        dst_p = dst_p.at[t, :cnt].set(flat_dst[row:row + cnt])
        row += cnt
    zeros = jnp.zeros((N_PAD, H), jnp.float32)

    s0 = _tc_matmul(x, W0)
    p0 = _segsum_sc(s0, src_p, dst_p, zeros)
    s1 = _tc_mid(p0, W1)
    p1 = _segsum_sc(s1, src_p, dst_p, zeros)
    return _tc_head(p1, eps, Wm1, bm1, Wm2, bm2,
                    Wl1, bl1, Wl2, bl2)


# speedup vs baseline: 2.5263x; 1.0006x over previous
import functools

import jax
import jax.numpy as jnp
from jax import lax
from jax.experimental import pallas as pl
from jax.experimental.pallas import tpu as pltpu
from jax.experimental.pallas import tpu_sc as plsc

N = 10000
E = 320000
D_IN = 128
H = 64
OUT = 32

NW = 32
LN = 128
GS = 3
NG0 = 31
NG1 = 23
K0 = NG0 * GS
K1 = NG1 * GS
TOTCH = 16 * (K0 + K1)
E_PAD = TOTCH * LN
N_PAD = 10112
RPT = N_PAD // 16

_mesh = plsc.VectorSubcoreMesh(core_axis_name="c", subcore_axis_name="s")


@functools.partial(
    pl.kernel,
    mesh=_mesh,
    compiler_params=pltpu.CompilerParams(use_tc_tiling_on_sc=False),
    out_type=jax.ShapeDtypeStruct((2, N_PAD, H), jnp.float32),
    scratch_types=[
        pltpu.VMEM((K0, LN), jnp.int32),
        pltpu.VMEM((K0, LN), jnp.int32),
        pltpu.VMEM((GS, LN, H), jnp.float32),
        pltpu.VMEM_SHARED((N_PAD, H), jnp.float32),
        pltpu.VMEM_SHARED((N_PAD, H), jnp.float32),
        pltpu.SemaphoreType.DMA((GS,)),
        pltpu.SemaphoreType.DMA,
    ],
)
def _segsum_sc(rows_hbm, src_hbm, dst_hbm, zeros_hbm, out_hbm,
               src_v, dst_v, buf_v, acc_sh, tbl_sh, gsem, ssem):
    c = lax.axis_index("c")
    s = lax.axis_index("s")

    def body(g, carry):
        base = g * GS
        gathers = [
            pltpu.async_copy(tbl_sh.at[src_v.at[base + b]], buf_v.at[b],
                             gsem.at[b])
            for b in range(GS)
        ]
        scatters = []
        for b in range(GS):
            gathers[b].wait()
            scatters.append(
                pltpu.async_copy(buf_v.at[b], acc_sh.at[dst_v.at[base + b]],
                                 ssem, add=True))
        for sc in scatters:
            sc.wait()
        return carry

    with jax.named_scope("segsum_stage"):
        wid = s * 2 + c
        stages = [
            pltpu.async_copy(rows_hbm.at[pl.ds(s * RPT, RPT)],
                             tbl_sh.at[pl.ds(s * RPT, RPT)], ssem),
            pltpu.async_copy(zeros_hbm.at[pl.ds(s * RPT, RPT)],
                             acc_sh.at[pl.ds(s * RPT, RPT)], ssem),
            pltpu.async_copy(src_hbm.at[wid], src_v, ssem),
            pltpu.async_copy(dst_hbm.at[wid], dst_v, ssem),
        ]
        for st in stages:
            st.wait()
        plsc.subcore_barrier()

    with jax.named_scope("segsum_edges"):
        @pl.when(c == 0)
        def _():
            lax.fori_loop(0, NG0, body, 0)

        @pl.when(c == 1)
        def _():
            lax.fori_loop(0, NG1, body, 0)

        plsc.subcore_barrier()

    with jax.named_scope("segsum_drain"):
        pltpu.sync_copy(acc_sh.at[pl.ds(s * RPT, RPT)],
                        out_hbm.at[c, pl.ds(s * RPT, RPT)])


def _mm_body(x_ref, w_ref, o_ref):
    o_ref[...] = jnp.dot(x_ref[...], w_ref[...],
                         preferred_element_type=jnp.float32)


def _mid_body(p_ref, w_ref, o_ref):
    m = p_ref[0] + p_ref[1]
    n = jnp.sqrt(jnp.sum(m * m, axis=1, keepdims=True))
    h = jnp.maximum(m / jnp.maximum(n, 1e-12), 0.0)
    o_ref[...] = jnp.dot(h, w_ref[...], preferred_element_type=jnp.float32)


def _head_body(p_ref, eps_ref, wm1_ref, bm1_ref, wm2_ref, bm2_ref,
               wl1_ref, bl1_ref, wl2_ref, bl2_ref, o_ref):
    m = p_ref[0] + p_ref[1]
    n = jnp.sqrt(jnp.sum(m * m, axis=1, keepdims=True))
    h = jnp.maximum(m / jnp.maximum(n, 1e-12), 0.0)
    t1 = jnp.tanh(jnp.dot(h, wm1_ref[...], preferred_element_type=jnp.float32)
                  + bm1_ref[...])
    mu = jnp.dot(t1, wm2_ref[...], preferred_element_type=jnp.float32) \
        + bm2_ref[...]
    t2 = jnp.tanh(jnp.dot(h, wl1_ref[...], preferred_element_type=jnp.float32)
                  + bl1_ref[...])
    ls = jnp.dot(t2, wl2_ref[...], preferred_element_type=jnp.float32) \
        + bl2_ref[...]
    ls = jnp.minimum(ls, 10.0)
    o_ref[...] = mu + eps_ref[...] * jnp.exp(ls)


_BM = 1000
_BMP = 632


def _tc_matmul(x, w):
    return pl.pallas_call(
        _mm_body,
        grid=(N_PAD // _BMP,),
        in_specs=[
            pl.BlockSpec((_BMP, x.shape[1]), lambda i: (i, 0)),
            pl.BlockSpec(w.shape, lambda i: (0, 0)),
        ],
        out_specs=pl.BlockSpec((_BMP, w.shape[1]), lambda i: (i, 0)),
        out_shape=jax.ShapeDtypeStruct((N_PAD, w.shape[1]), jnp.float32),
    )(x, w)


def _tc_mid(p, w):
    return pl.pallas_call(
        _mid_body,
        grid=(N_PAD // _BMP,),
        in_specs=[
            pl.BlockSpec((2, _BMP, H), lambda i: (0, i, 0)),
            pl.BlockSpec((H, H), lambda i: (0, 0)),
        ],
        out_specs=pl.BlockSpec((_BMP, H), lambda i: (i, 0)),
        out_shape=jax.ShapeDtypeStruct((N_PAD, H), jnp.float32),
    )(p, w)


def _tc_head(p, eps, wm1, bm1, wm2, bm2, wl1, bl1, wl2, bl2):
    full = lambda shape: pl.BlockSpec(shape, lambda i: tuple(0 for _ in shape))
    return pl.pallas_call(
        _head_body,
        grid=(N // _BM,),
        in_specs=[
            pl.BlockSpec((2, _BM, H), lambda i: (0, i, 0)),
            pl.BlockSpec((_BM, OUT), lambda i: (i, 0)),
            full(wm1.shape), full((1, OUT // 2)),
            full(wm2.shape), full((1, OUT)),
            full(wl1.shape), full((1, OUT // 2)),
            full(wl2.shape), full((1, OUT)),
        ],
        out_specs=pl.BlockSpec((_BM, OUT), lambda i: (i, 0)),
        out_shape=jax.ShapeDtypeStruct((N, OUT), jnp.float32),
    )(p, eps, wm1, bm1.reshape(1, -1), wm2, bm2.reshape(1, -1),
      wl1, bl1.reshape(1, -1), wl2, bl2.reshape(1, -1))


def kernel(x, edge_index, eps, W0, W1, Wm1, bm1, Wm2, bm2, Wl1, bl1, Wl2, bl2):
    src = edge_index[0].astype(jnp.int32)
    dst = edge_index[1].astype(jnp.int32)
    pad = E_PAD - E
    flat_src = jnp.concatenate([src, jnp.zeros((pad,), jnp.int32)])
    flat_dst = jnp.concatenate([dst, jnp.full((pad,), N, jnp.int32)])
    flat_src = flat_src.reshape(TOTCH, LN)
    flat_dst = flat_dst.reshape(TOTCH, LN)

    def _tile_rows(flat, fill):
        a = flat[:16 * K0].reshape(16, K0, LN)
        b = jnp.pad(flat[16 * K0:].reshape(16, K1, LN),
                    ((0, 0), (0, K0 - K1), (0, 0)), constant_values=fill)
        return jnp.stack([a, b], axis=1).reshape(NW, K0, LN)

    src_p = _tile_rows(flat_src, 0)
    dst_p = _tile_rows(flat_dst, N)
    zeros = jnp.zeros((N_PAD, H), jnp.float32)

    s0 = _tc_matmul(x, W0)
    p0 = _segsum_sc(s0, src_p, dst_p, zeros)
    s1 = _tc_mid(p0, W1)
    p1 = _segsum_sc(s1, src_p, dst_p, zeros)
    return _tc_head(p1, eps, Wm1, bm1, Wm2, bm2,
                    Wl1, bl1, Wl2, bl2)
